# baseline (speedup 1.0000x reference)
import functools

import jax
import jax.numpy as jnp
from jax import lax
from jax.experimental import pallas as pl
from jax.experimental.pallas import tpu as pltpu
from jax.experimental.pallas import tpu_sc as plsc

NUM_EMB = 1000000
DIM = 64
BATCH = 4096
FIELDS = 26
TOTAL = BATCH * FIELDS

NC = 2
NS = 16
NW = NC * NS
PER_W = TOTAL // NW
CHUNK = 128
NCHUNK = PER_W // CHUNK
NBUF = 4

_mesh = plsc.VectorSubcoreMesh(core_axis_name="c", subcore_axis_name="s")


@functools.partial(
    pl.kernel,
    mesh=_mesh,
    out_type=jax.ShapeDtypeStruct((TOTAL, DIM), jnp.float32),
    compiler_params=pltpu.CompilerParams(use_tc_tiling_on_sc=False),
    scratch_types=[
        pltpu.VMEM((PER_W,), jnp.int32),
        pltpu.VMEM((NBUF, CHUNK, DIM), jnp.float32),
        pltpu.SemaphoreType.DMA,
        pltpu.SemaphoreType.DMA,
    ],
)
def _gather_kernel(idx_hbm, tab_hbm, out_hbm, idx_v, rows_v, gsem, wsem):
    wid = lax.axis_index("s") * NC + lax.axis_index("c")
    base = wid * PER_W

    pltpu.sync_copy(idx_hbm.at[pl.ds(base, PER_W)], idx_v)

    gathers = [None] * NCHUNK
    writes = [None] * NCHUNK
    for j in range(min(NBUF, NCHUNK)):
        gathers[j] = pltpu.async_copy(
            tab_hbm.at[idx_v.at[pl.ds(j * CHUNK, CHUNK)]],
            rows_v.at[j % NBUF], gsem)
    for j in range(NCHUNK):
        gathers[j].wait()
        writes[j] = pltpu.async_copy(
            rows_v.at[j % NBUF],
            out_hbm.at[pl.ds(base + j * CHUNK, CHUNK)],
            wsem)
        nxt = j + NBUF
        if nxt < NCHUNK:
            writes[j].wait()
            gathers[nxt] = pltpu.async_copy(
                tab_hbm.at[idx_v.at[pl.ds(nxt * CHUNK, CHUNK)]],
                rows_v.at[nxt % NBUF], gsem)
    for j in range(max(0, NCHUNK - NBUF), NCHUNK):
        writes[j].wait()


def kernel(x, weight):
    idx = x.astype(jnp.int32).reshape(TOTAL)
    out = _gather_kernel(idx, weight)
    return out.reshape(BATCH, FIELDS, DIM)
